# Initial kernel scaffold; baseline (speedup 1.0000x reference)
import jax
import jax.numpy as jnp
from jax.experimental import pallas as pl


def kernel(x, codebook):
    raise NotImplementedError("write your pallas kernel here")


# speedup vs baseline: 2.5750x; 2.5750x over previous
import jax
import jax.numpy as jnp
from jax.experimental import pallas as pl
from jax.experimental.pallas import tpu as pltpu

_D = 256
_K = 1024
_L = 96
_B = 64
_B_BLK = 4
_SCALE = 1.25 / (_B * _L * _D)


def _vq_body(x_ref, cb_ref, q_ref, loss_ref):
    i = pl.program_id(0)
    cb = cb_ref[...]
    xcat = jnp.concatenate([x_ref[b][:, :_L] for b in range(_B_BLK)], axis=1)
    ip = jnp.dot(cb, xcat, preferred_element_type=jnp.float32)
    c2 = jnp.sum(cb * cb, axis=1, keepdims=True)
    dist = c2 - 2.0 * ip
    idx = jnp.argmin(dist, axis=0)
    onehot = (jax.lax.broadcasted_iota(jnp.int32, dist.shape, 0)
              == idx[None, :]).astype(jnp.float32)
    q = jax.lax.dot_general(cb, onehot, (((0,), (0,)), ((), ())),
                            preferred_element_type=jnp.float32)
    for b in range(_B_BLK):
        q_ref[b] = q[:, b * _L:(b + 1) * _L]
    diff = q - xcat
    part = jnp.sum(diff * diff)

    @pl.when(i == 0)
    def _init():
        loss_ref[0, 0] = part

    @pl.when(i > 0)
    def _acc():
        loss_ref[0, 0] += part

    @pl.when(i == (_B // _B_BLK) - 1)
    def _final():
        loss_ref[0, 0] *= _SCALE


def kernel(x, codebook):
    q, loss = pl.pallas_call(
        _vq_body,
        grid=(_B // _B_BLK,),
        in_specs=[
            pl.BlockSpec((_B_BLK, _D, 128), lambda i: (i, 0, 0)),
            pl.BlockSpec((_K, _D), lambda i: (0, 0)),
        ],
        out_specs=[
            pl.BlockSpec((_B_BLK, _D, _L), lambda i: (i, 0, 0)),
            pl.BlockSpec(memory_space=pltpu.SMEM),
        ],
        out_shape=[
            jax.ShapeDtypeStruct((_B, _D, _L), jnp.float32),
            jax.ShapeDtypeStruct((1, 1), jnp.float32),
        ],
    )(x, codebook)
    return q, loss[0, 0]


# speedup vs baseline: 2.6955x; 1.0468x over previous
import jax
import jax.numpy as jnp
from jax.experimental import pallas as pl
from jax.experimental.pallas import tpu as pltpu

_D = 256
_K = 1024
_L = 96
_B = 64
_B_BLK = 4
_N = _B_BLK * 128
_SCALE = 1.25 / (_B * _L * _D)


def _vq_body(x_ref, cb_ref, q_ref, loss_ref, c2_ref):
    i = pl.program_id(0)
    cb = cb_ref[...]

    @pl.when(i == 0)
    def _c2():
        c2 = jnp.sum(cb * cb, axis=1, keepdims=True)
        c2_ref[...] = 0.5 * jnp.broadcast_to(c2, (_K, _N))

    xcat = jnp.concatenate([x_ref[b] for b in range(_B_BLK)], axis=1)
    ip = jnp.dot(cb, xcat, preferred_element_type=jnp.float32)
    dist = c2_ref[...] - ip
    idx = jnp.argmin(dist, axis=0)
    onehot = (jax.lax.broadcasted_iota(jnp.int32, (_K, _N), 0)
              == idx[None, :]).astype(jnp.float32)
    q = jax.lax.dot_general(cb, onehot, (((0,), (0,)), ((), ())),
                            preferred_element_type=jnp.float32)
    part = None
    for b in range(_B_BLK):
        qb = q[:, b * 128:b * 128 + _L]
        q_ref[b] = qb
        db = qb - x_ref[b][:, :_L]
        p = jnp.sum(db * db)
        part = p if part is None else part + p

    @pl.when(i == 0)
    def _init():
        loss_ref[0, 0] = part

    @pl.when(i > 0)
    def _acc():
        loss_ref[0, 0] += part

    @pl.when(i == (_B // _B_BLK) - 1)
    def _final():
        loss_ref[0, 0] *= _SCALE


def kernel(x, codebook):
    q, loss = pl.pallas_call(
        _vq_body,
        grid=(_B // _B_BLK,),
        in_specs=[
            pl.BlockSpec((_B_BLK, _D, 128), lambda i: (i, 0, 0)),
            pl.BlockSpec((_K, _D), lambda i: (0, 0)),
        ],
        out_specs=[
            pl.BlockSpec((_B_BLK, _D, _L), lambda i: (i, 0, 0)),
            pl.BlockSpec(memory_space=pltpu.SMEM),
        ],
        out_shape=[
            jax.ShapeDtypeStruct((_B, _D, _L), jnp.float32),
            jax.ShapeDtypeStruct((1, 1), jnp.float32),
        ],
        scratch_shapes=[pltpu.VMEM((_K, _N), jnp.float32)],
    )(x, codebook)
    return q, loss[0, 0]


# speedup vs baseline: 2.7045x; 1.0034x over previous
import jax
import jax.numpy as jnp
from jax.experimental import pallas as pl
from jax.experimental.pallas import tpu as pltpu

_D = 256
_K = 1024
_L = 96
_B = 64
_B_BLK = 4
_N = _B_BLK * 128
_SCALE = 1.25 / (_B * _L * _D)


def _vq_body(x_ref, cb_ref, q_ref, loss_ref, c2_ref, cb16_ref):
    i = pl.program_id(0)
    cb = cb_ref[...]

    @pl.when(i == 0)
    def _c2():
        c2 = jnp.sum(cb * cb, axis=1, keepdims=True)
        c2_ref[...] = 0.5 * jnp.broadcast_to(c2, (_K, _N))
        cb16_ref[...] = cb.astype(jnp.bfloat16)

    xcat = jnp.concatenate([x_ref[b] for b in range(_B_BLK)], axis=1)
    ip = jnp.dot(cb, xcat, preferred_element_type=jnp.float32)
    dist = c2_ref[...] - ip
    idx = jnp.argmin(dist, axis=0)
    onehot = (jax.lax.broadcasted_iota(jnp.int32, (_K, _N), 0)
              == idx[None, :]).astype(jnp.bfloat16)
    q = jax.lax.dot_general(cb16_ref[...], onehot, (((0,), (0,)), ((), ())),
                            preferred_element_type=jnp.float32)
    part = None
    for b in range(_B_BLK):
        qb = q[:, b * 128:b * 128 + _L]
        q_ref[b] = qb
        db = qb - x_ref[b][:, :_L]
        p = jnp.sum(db * db)
        part = p if part is None else part + p

    @pl.when(i == 0)
    def _init():
        loss_ref[0, 0] = part

    @pl.when(i > 0)
    def _acc():
        loss_ref[0, 0] += part

    @pl.when(i == (_B // _B_BLK) - 1)
    def _final():
        loss_ref[0, 0] *= _SCALE


def kernel(x, codebook):
    q, loss = pl.pallas_call(
        _vq_body,
        grid=(_B // _B_BLK,),
        in_specs=[
            pl.BlockSpec((_B_BLK, _D, 128), lambda i: (i, 0, 0)),
            pl.BlockSpec((_K, _D), lambda i: (0, 0)),
        ],
        out_specs=[
            pl.BlockSpec((_B_BLK, _D, _L), lambda i: (i, 0, 0)),
            pl.BlockSpec(memory_space=pltpu.SMEM),
        ],
        out_shape=[
            jax.ShapeDtypeStruct((_B, _D, _L), jnp.float32),
            jax.ShapeDtypeStruct((1, 1), jnp.float32),
        ],
        scratch_shapes=[pltpu.VMEM((_K, _N), jnp.float32),
                        pltpu.VMEM((_K, _D), jnp.bfloat16)],
    )(x, codebook)
    return q, loss[0, 0]


# speedup vs baseline: 3.2759x; 1.2113x over previous
import jax
import jax.numpy as jnp
from jax.experimental import pallas as pl
from jax.experimental.pallas import tpu as pltpu

_D = 256
_K = 1024
_L = 96
_B = 64
_B_BLK = 8
_N = _B_BLK * 128
_SCALE = 1.25 / (_B * _L * _D)


def _vq_body(x_ref, cb_ref, q_ref, loss_ref, c2_ref, cb16_ref):
    i = pl.program_id(0)
    cb = cb_ref[...]

    @pl.when(i == 0)
    def _c2():
        c2 = jnp.sum(cb * cb, axis=1, keepdims=True)
        c2_ref[...] = 0.5 * jnp.broadcast_to(c2, (_K, _N))
        cb16_ref[...] = cb.astype(jnp.bfloat16)

    xcat = jnp.concatenate([x_ref[b] for b in range(_B_BLK)], axis=1)
    ip = jnp.dot(cb, xcat, preferred_element_type=jnp.float32)
    dist = c2_ref[...] - ip
    idx = jnp.argmin(dist, axis=0)
    onehot = (jax.lax.broadcasted_iota(jnp.int32, (_K, _N), 0)
              == idx[None, :]).astype(jnp.bfloat16)
    q = jax.lax.dot_general(cb16_ref[...], onehot, (((0,), (0,)), ((), ())),
                            preferred_element_type=jnp.float32)
    part = None
    for b in range(_B_BLK):
        qb = q[:, b * 128:b * 128 + _L]
        q_ref[b] = qb
        db = qb - x_ref[b][:, :_L]
        p = jnp.sum(db * db)
        part = p if part is None else part + p

    @pl.when(i == 0)
    def _init():
        loss_ref[0, 0] = part

    @pl.when(i > 0)
    def _acc():
        loss_ref[0, 0] += part

    @pl.when(i == (_B // _B_BLK) - 1)
    def _final():
        loss_ref[0, 0] *= _SCALE


def kernel(x, codebook):
    q, loss = pl.pallas_call(
        _vq_body,
        grid=(_B // _B_BLK,),
        in_specs=[
            pl.BlockSpec((_B_BLK, _D, 128), lambda i: (i, 0, 0)),
            pl.BlockSpec((_K, _D), lambda i: (0, 0)),
        ],
        out_specs=[
            pl.BlockSpec((_B_BLK, _D, _L), lambda i: (i, 0, 0)),
            pl.BlockSpec(memory_space=pltpu.SMEM),
        ],
        out_shape=[
            jax.ShapeDtypeStruct((_B, _D, _L), jnp.float32),
            jax.ShapeDtypeStruct((1, 1), jnp.float32),
        ],
        scratch_shapes=[pltpu.VMEM((_K, _N), jnp.float32),
                        pltpu.VMEM((_K, _D), jnp.bfloat16)],
    )(x, codebook)
    return q, loss[0, 0]


# speedup vs baseline: 3.5042x; 1.0697x over previous
import jax
import jax.numpy as jnp
from jax.experimental import pallas as pl
from jax.experimental.pallas import tpu as pltpu

_D = 256
_K = 1024
_L = 96
_B = 64
_B_BLK = 16
_N = _B_BLK * 128
_SCALE = 1.25 / (_B * _L * _D)


def _vq_body(x_ref, cb_ref, q_ref, loss_ref, c2_ref, cb16_ref):
    i = pl.program_id(0)
    cb = cb_ref[...]

    @pl.when(i == 0)
    def _c2():
        c2 = jnp.sum(cb * cb, axis=1, keepdims=True)
        c2_ref[...] = 0.5 * jnp.broadcast_to(c2, (_K, _N))
        cb16_ref[...] = cb.astype(jnp.bfloat16)

    xcat = jnp.concatenate([x_ref[b] for b in range(_B_BLK)], axis=1)
    ip = jnp.dot(cb, xcat, preferred_element_type=jnp.float32)
    dist = c2_ref[...] - ip
    idx = jnp.argmin(dist, axis=0)
    onehot = (jax.lax.broadcasted_iota(jnp.int32, (_K, _N), 0)
              == idx[None, :]).astype(jnp.bfloat16)
    q = jax.lax.dot_general(cb16_ref[...], onehot, (((0,), (0,)), ((), ())),
                            preferred_element_type=jnp.float32)
    part = None
    for b in range(_B_BLK):
        qb = q[:, b * 128:b * 128 + _L]
        q_ref[b] = qb
        db = qb - x_ref[b][:, :_L]
        p = jnp.sum(db * db)
        part = p if part is None else part + p

    @pl.when(i == 0)
    def _init():
        loss_ref[0, 0] = part

    @pl.when(i > 0)
    def _acc():
        loss_ref[0, 0] += part

    @pl.when(i == (_B // _B_BLK) - 1)
    def _final():
        loss_ref[0, 0] *= _SCALE


def kernel(x, codebook):
    q, loss = pl.pallas_call(
        _vq_body,
        grid=(_B // _B_BLK,),
        in_specs=[
            pl.BlockSpec((_B_BLK, _D, 128), lambda i: (i, 0, 0)),
            pl.BlockSpec((_K, _D), lambda i: (0, 0)),
        ],
        out_specs=[
            pl.BlockSpec((_B_BLK, _D, _L), lambda i: (i, 0, 0)),
            pl.BlockSpec(memory_space=pltpu.SMEM),
        ],
        out_shape=[
            jax.ShapeDtypeStruct((_B, _D, _L), jnp.float32),
            jax.ShapeDtypeStruct((1, 1), jnp.float32),
        ],
        scratch_shapes=[pltpu.VMEM((_K, _N), jnp.float32),
                        pltpu.VMEM((_K, _D), jnp.bfloat16)],
    )(x, codebook)
    return q, loss[0, 0]


# speedup vs baseline: 3.5198x; 1.0045x over previous
import jax
import jax.numpy as jnp
from jax.experimental import pallas as pl
from jax.experimental.pallas import tpu as pltpu

_D = 256
_K = 1024
_L = 96
_B = 64
_B_BLK = 32
_N = _B_BLK * 128
_SCALE = 1.25 / (_B * _L * _D)


def _vq_body(x_ref, cb_ref, q_ref, loss_ref, c2_ref, cb16_ref):
    i = pl.program_id(0)
    cb = cb_ref[...]

    @pl.when(i == 0)
    def _c2():
        c2 = jnp.sum(cb * cb, axis=1, keepdims=True)
        c2_ref[...] = 0.5 * jnp.broadcast_to(c2, (_K, 128))
        cb16_ref[...] = cb.astype(jnp.bfloat16)

    xcat = jnp.concatenate([x_ref[b] for b in range(_B_BLK)], axis=1)
    ip = jnp.dot(cb, xcat, preferred_element_type=jnp.float32)
    dist = c2_ref[:, :1] - ip
    idx = jnp.argmin(dist, axis=0)
    onehot = (jax.lax.broadcasted_iota(jnp.int32, (_K, _N), 0)
              == idx[None, :]).astype(jnp.bfloat16)
    q = jax.lax.dot_general(cb16_ref[...], onehot, (((0,), (0,)), ((), ())),
                            preferred_element_type=jnp.float32)
    part = None
    for b in range(_B_BLK):
        qb = q[:, b * 128:b * 128 + _L]
        q_ref[b] = qb
        db = qb - x_ref[b][:, :_L]
        p = jnp.sum(db * db)
        part = p if part is None else part + p

    @pl.when(i == 0)
    def _init():
        loss_ref[0, 0] = part

    @pl.when(i > 0)
    def _acc():
        loss_ref[0, 0] += part

    @pl.when(i == (_B // _B_BLK) - 1)
    def _final():
        loss_ref[0, 0] *= _SCALE


def kernel(x, codebook):
    q, loss = pl.pallas_call(
        _vq_body,
        grid=(_B // _B_BLK,),
        in_specs=[
            pl.BlockSpec((_B_BLK, _D, 128), lambda i: (i, 0, 0)),
            pl.BlockSpec((_K, _D), lambda i: (0, 0)),
        ],
        out_specs=[
            pl.BlockSpec((_B_BLK, _D, _L), lambda i: (i, 0, 0)),
            pl.BlockSpec(memory_space=pltpu.SMEM),
        ],
        out_shape=[
            jax.ShapeDtypeStruct((_B, _D, _L), jnp.float32),
            jax.ShapeDtypeStruct((1, 1), jnp.float32),
        ],
        scratch_shapes=[pltpu.VMEM((_K, 128), jnp.float32),
                        pltpu.VMEM((_K, _D), jnp.bfloat16)],
    )(x, codebook)
    return q, loss[0, 0]


# speedup vs baseline: 4.3879x; 1.2466x over previous
import jax
import jax.numpy as jnp
from jax.experimental import pallas as pl
from jax.experimental.pallas import tpu as pltpu

_D = 256
_K = 1024
_L = 96
_B = 64
_B_BLK = 32
_N = _B_BLK * 128
_SCALE = 1.25 / (_B * _L * _D)


def _vq_body(x_ref, cb_ref, q_ref, loss_ref, c2_ref, cb16_ref):
    i = pl.program_id(0)
    cb = cb_ref[...]

    @pl.when(i == 0)
    def _prep():
        c2 = jnp.sum(cb * cb, axis=1, keepdims=True)
        c2_ref[...] = 0.5 * jnp.broadcast_to(c2, (_K, 128))
        cb16_ref[...] = cb.astype(jnp.bfloat16)

    xcat = jnp.concatenate([x_ref[b] for b in range(_B_BLK)], axis=1)
    ip = jnp.dot(cb, xcat, preferred_element_type=jnp.float32)
    dist = c2_ref[:, :1] - ip
    idx = jnp.argmin(dist, axis=0)
    onehot = (jax.lax.broadcasted_iota(jnp.int32, (_K, _N), 0)
              == idx[None, :]).astype(jnp.bfloat16)
    q = jax.lax.dot_general(onehot, cb16_ref[...], (((0,), (0,)), ((), ())),
                            preferred_element_type=jnp.float32)
    for b in range(_B_BLK):
        q_ref[b] = q[b * 128:b * 128 + _L, :]

    x2 = jnp.sum(xcat * xcat, axis=0, keepdims=True)
    mind = jnp.min(dist, axis=0, keepdims=True)
    lane = jax.lax.broadcasted_iota(jnp.int32, (1, _N), 1)
    used = (lane % 128) < _L
    part = jnp.sum(jnp.where(used, x2 + 2.0 * mind, 0.0))

    @pl.when(i == 0)
    def _init():
        loss_ref[0, 0] = part

    @pl.when(i > 0)
    def _acc():
        loss_ref[0, 0] += part

    @pl.when(i == (_B // _B_BLK) - 1)
    def _final():
        loss_ref[0, 0] *= _SCALE


def kernel(x, codebook):
    q, loss = pl.pallas_call(
        _vq_body,
        grid=(_B // _B_BLK,),
        in_specs=[
            pl.BlockSpec((_B_BLK, _D, 128), lambda i: (i, 0, 0)),
            pl.BlockSpec((_K, _D), lambda i: (0, 0)),
        ],
        out_specs=[
            pl.BlockSpec((_B_BLK, _L, _D), lambda i: (i, 0, 0)),
            pl.BlockSpec(memory_space=pltpu.SMEM),
        ],
        out_shape=[
            jax.ShapeDtypeStruct((_B, _L, _D), jnp.float32),
            jax.ShapeDtypeStruct((1, 1), jnp.float32),
        ],
        scratch_shapes=[pltpu.VMEM((_K, 128), jnp.float32),
                        pltpu.VMEM((_K, _D), jnp.bfloat16)],
    )(x, codebook)
    return jnp.transpose(q, (0, 2, 1)), loss[0, 0]


# speedup vs baseline: 5.3216x; 1.2128x over previous
import jax
import jax.numpy as jnp
from jax.experimental import pallas as pl
from jax.experimental.pallas import tpu as pltpu

_D = 256
_K = 1024
_L = 96
_B = 64
_B_BLK = 32
_N = _B_BLK * _L
_SCALE = 1.25 / (_B * _L * _D)


def _vq_body(x_ref, cb_ref, q_ref, loss_ref, c2_ref, cb16_ref):
    i = pl.program_id(0)
    cb = cb_ref[...]

    @pl.when(i == 0)
    def _prep():
        c2 = jnp.sum(cb * cb, axis=1, keepdims=True)
        c2_ref[...] = 0.5 * jnp.broadcast_to(c2, (_K, 128))
        cb16_ref[...] = cb.astype(jnp.bfloat16)

    xcat = jnp.concatenate([x_ref[b][:, :_L] for b in range(_B_BLK)], axis=1)
    ip = jnp.dot(cb, xcat, preferred_element_type=jnp.float32)
    dist = c2_ref[:, :1] - ip
    idx = jnp.argmin(dist, axis=0)
    onehot = (jax.lax.broadcasted_iota(jnp.int32, (_K, _N), 0)
              == idx[None, :]).astype(jnp.bfloat16)
    q = jax.lax.dot_general(onehot, cb16_ref[...], (((0,), (0,)), ((), ())),
                            preferred_element_type=jnp.float32)
    for b in range(_B_BLK):
        q_ref[b] = q[b * _L:(b + 1) * _L, :]

    x2 = jnp.sum(xcat * xcat, axis=0, keepdims=True)
    mind = jnp.min(dist, axis=0, keepdims=True)
    part = jnp.sum(x2 + 2.0 * mind)

    @pl.when(i == 0)
    def _init():
        loss_ref[0, 0] = part

    @pl.when(i > 0)
    def _acc():
        loss_ref[0, 0] += part

    @pl.when(i == (_B // _B_BLK) - 1)
    def _final():
        loss_ref[0, 0] *= _SCALE


def kernel(x, codebook):
    q, loss = pl.pallas_call(
        _vq_body,
        grid=(_B // _B_BLK,),
        in_specs=[
            pl.BlockSpec((_B_BLK, _D, 128), lambda i: (i, 0, 0)),
            pl.BlockSpec((_K, _D), lambda i: (0, 0)),
        ],
        out_specs=[
            pl.BlockSpec((_B_BLK, _L, _D), lambda i: (i, 0, 0)),
            pl.BlockSpec(memory_space=pltpu.SMEM),
        ],
        out_shape=[
            jax.ShapeDtypeStruct((_B, _L, _D), jnp.float32),
            jax.ShapeDtypeStruct((1, 1), jnp.float32),
        ],
        scratch_shapes=[pltpu.VMEM((_K, 128), jnp.float32),
                        pltpu.VMEM((_K, _D), jnp.bfloat16)],
    )(x, codebook)
    return jnp.transpose(q, (0, 2, 1)), loss[0, 0]
